# baseline (speedup 1.0000x reference)
import jax
import jax.numpy as jnp
from jax.experimental import pallas as pl
from jax.experimental.pallas import tpu as pltpu

_N = 10000
_NP = 10112
_H = 2048
_D = 128

_HB = 512
_NB = 2000


def _mm_rows_kernel(a_ref, b_ref, o_ref):
    o_ref[...] = jnp.dot(a_ref[...], b_ref[...].astype(jnp.bfloat16),
                         preferred_element_type=jnp.float32)


def _layer_out_kernel(src_ref, t_ref, x_ref, o_ref):
    y = jnp.dot(src_ref[...].astype(jnp.bfloat16),
                t_ref[...].astype(jnp.bfloat16),
                preferred_element_type=jnp.float32)
    o_ref[...] = jnp.maximum(y, 0.0) + x_ref[...]


def _final_kernel(w_ref, src_ref, t_ref, x0_ref, x1_ref, o_ref):
    y = jnp.dot(src_ref[...].astype(jnp.bfloat16),
                t_ref[...].astype(jnp.bfloat16),
                preferred_element_type=jnp.float32)
    x1 = x1_ref[...]
    x2 = jnp.maximum(y, 0.0) + x1
    o_ref[...] = w_ref[0] * x0_ref[...] + w_ref[1] * x1 + w_ref[2] * x2


def _mm_rows(a16, bp):
    return pl.pallas_call(
        _mm_rows_kernel,
        grid=(_H // _HB,),
        in_specs=[
            pl.BlockSpec((_HB, _NP), lambda i: (i, 0)),
            pl.BlockSpec((_NP, _D), lambda i: (0, 0)),
        ],
        out_specs=pl.BlockSpec((_HB, _D), lambda i: (i, 0)),
        out_shape=jax.ShapeDtypeStruct((_H, _D), jnp.float32),
    )(a16, bp)


def kernel(pois_embs, HG_poi_src, HG_poi_tar, layer_attention):
    w = jax.nn.softmax(layer_attention, axis=0)

    tar16 = jnp.pad(HG_poi_tar.astype(jnp.bfloat16),
                    ((0, 0), (0, _NP - _N)))
    x0p = jnp.pad(pois_embs, ((0, _NP - _N), (0, 0)))

    t1 = _mm_rows(tar16, x0p)
    x1 = pl.pallas_call(
        _layer_out_kernel,
        grid=(_N // _NB,),
        in_specs=[
            pl.BlockSpec((_NB, _H), lambda i: (i, 0)),
            pl.BlockSpec((_H, _D), lambda i: (0, 0)),
            pl.BlockSpec((_NB, _D), lambda i: (i, 0)),
        ],
        out_specs=pl.BlockSpec((_NB, _D), lambda i: (i, 0)),
        out_shape=jax.ShapeDtypeStruct((_N, _D), jnp.float32),
    )(HG_poi_src, t1, pois_embs)

    x1p = jnp.pad(x1, ((0, _NP - _N), (0, 0)))
    t2 = _mm_rows(tar16, x1p)
    out = pl.pallas_call(
        _final_kernel,
        grid=(_N // _NB,),
        in_specs=[
            pl.BlockSpec(memory_space=pltpu.SMEM),
            pl.BlockSpec((_NB, _H), lambda i: (i, 0)),
            pl.BlockSpec((_H, _D), lambda i: (0, 0)),
            pl.BlockSpec((_NB, _D), lambda i: (i, 0)),
            pl.BlockSpec((_NB, _D), lambda i: (i, 0)),
        ],
        out_specs=pl.BlockSpec((_NB, _D), lambda i: (i, 0)),
        out_shape=jax.ShapeDtypeStruct((_N, _D), jnp.float32),
    )(w, HG_poi_src, t2, pois_embs, x1)
    return out


# speedup vs baseline: 1.5000x; 1.5000x over previous
import jax
import jax.numpy as jnp
from jax.experimental import pallas as pl
from jax.experimental.pallas import tpu as pltpu

_N = 10000
_H = 2048
_D = 128

_TB = 2000
_NB = 1000


def _t1_kernel(tarT_ref, x_ref, t_ref):
    j = pl.program_id(0)

    @pl.when(j == 0)
    def _():
        t_ref[...] = jnp.zeros_like(t_ref)

    t_ref[...] += jax.lax.dot_general(
        tarT_ref[...], x_ref[...].astype(jnp.bfloat16),
        (((0,), (0,)), ((), ())),
        preferred_element_type=jnp.float32)


def _fused_kernel(src_ref, tarT_ref, t1_ref, x0_ref, x1_ref, t2_ref):
    j = pl.program_id(0)
    y = jnp.dot(src_ref[...].astype(jnp.bfloat16),
                t1_ref[...].astype(jnp.bfloat16),
                preferred_element_type=jnp.float32)
    x1 = jnp.maximum(y, 0.0) + x0_ref[...]
    x1_ref[...] = x1

    @pl.when(j == 0)
    def _():
        t2_ref[...] = jnp.zeros_like(t2_ref)

    t2_ref[...] += jax.lax.dot_general(
        tarT_ref[...], x1.astype(jnp.bfloat16),
        (((0,), (0,)), ((), ())),
        preferred_element_type=jnp.float32)


def _final_kernel(w_ref, src_ref, t2_ref, x0_ref, x1_ref, o_ref):
    y = jnp.dot(src_ref[...].astype(jnp.bfloat16),
                t2_ref[...].astype(jnp.bfloat16),
                preferred_element_type=jnp.float32)
    x1 = x1_ref[...]
    x2 = jnp.maximum(y, 0.0) + x1
    o_ref[...] = w_ref[0] * x0_ref[...] + w_ref[1] * x1 + w_ref[2] * x2


def kernel(pois_embs, HG_poi_src, HG_poi_tar, layer_attention):
    w = jax.nn.softmax(layer_attention, axis=0)
    tarT16 = HG_poi_tar.T.astype(jnp.bfloat16)

    t1 = pl.pallas_call(
        _t1_kernel,
        grid=(_N // _TB,),
        in_specs=[
            pl.BlockSpec((_TB, _H), lambda j: (j, 0)),
            pl.BlockSpec((_TB, _D), lambda j: (j, 0)),
        ],
        out_specs=pl.BlockSpec((_H, _D), lambda j: (0, 0)),
        out_shape=jax.ShapeDtypeStruct((_H, _D), jnp.float32),
    )(tarT16, pois_embs)

    x1, t2 = pl.pallas_call(
        _fused_kernel,
        grid=(_N // _NB,),
        in_specs=[
            pl.BlockSpec((_NB, _H), lambda j: (j, 0)),
            pl.BlockSpec((_NB, _H), lambda j: (j, 0)),
            pl.BlockSpec((_H, _D), lambda j: (0, 0)),
            pl.BlockSpec((_NB, _D), lambda j: (j, 0)),
        ],
        out_specs=(
            pl.BlockSpec((_NB, _D), lambda j: (j, 0)),
            pl.BlockSpec((_H, _D), lambda j: (0, 0)),
        ),
        out_shape=(
            jax.ShapeDtypeStruct((_N, _D), jnp.float32),
            jax.ShapeDtypeStruct((_H, _D), jnp.float32),
        ),
    )(HG_poi_src, tarT16, t1, pois_embs)

    out = pl.pallas_call(
        _final_kernel,
        grid=(_N // _NB,),
        in_specs=[
            pl.BlockSpec(memory_space=pltpu.SMEM),
            pl.BlockSpec((_NB, _H), lambda j: (j, 0)),
            pl.BlockSpec((_H, _D), lambda j: (0, 0)),
            pl.BlockSpec((_NB, _D), lambda j: (j, 0)),
            pl.BlockSpec((_NB, _D), lambda j: (j, 0)),
        ],
        out_specs=pl.BlockSpec((_NB, _D), lambda j: (j, 0)),
        out_shape=jax.ShapeDtypeStruct((_N, _D), jnp.float32),
    )(w, HG_poi_src, t2, pois_embs, x1)
    return out
